# baseline (speedup 1.0000x reference)
import functools

import jax
import jax.numpy as jnp
from jax import lax
from jax.experimental import pallas as pl
from jax.experimental.pallas import tpu as pltpu
from jax.experimental.pallas import tpu_sc as plsc

N_ROWS = 160000
D = 256
NSEG = 10000
L = 16
DL = D // L
NW = 32
SPW = 320
SPW_LAST = NSEG - (NW - 1) * SPW
CHUNK = 80
MAXK = N_ROWS // CHUNK - 1
ROWB_PAD = 48


def _sc_body(feats_hbm, ids_hbm, rowb_hbm, wb_hbm, out_hbm,
             wb_v, rowb_v, ids0_v, ids1_v, f0_v, f1_v, acc_v,
             sf0, sf1, si0, si1, fsem):
    cid = lax.axis_index("c")
    sid = lax.axis_index("s")
    wid = sid * 2 + cid

    pltpu.sync_copy(rowb_hbm, rowb_v)
    pltpu.sync_copy(wb_hbm, wb_v)

    r_lo = rowb_v[pl.ds(wid, L)][0]
    r_hi = rowb_v[pl.ds(wid + 1, L)][0]
    seg_base = wid * SPW

    Ws = [wb_v[pl.ds(j * L, L)] for j in range(DL)]
    bsplat = wb_v[pl.ds(D, L)]

    zero = jnp.zeros((L,), jnp.float32)

    def zrow(r, carry):
        for j in range(DL):
            acc_v[r, pl.ds(j * L, L)] = zero
        return carry

    lax.fori_loop(0, SPW, zrow, 0)

    def handles(k, f_v, i_v, s_f, s_i):
        kc = jnp.minimum(k, MAXK)
        row0 = kc * CHUNK
        h_f = pltpu.make_async_copy(
            feats_hbm.at[pl.ds(row0, CHUNK)], f_v, s_f)
        h_i = pltpu.make_async_copy(
            ids_hbm.at[pl.ds(row0, CHUNK)], i_v.at[pl.ds(0, CHUNK)], s_i)
        return h_f, h_i

    def start(k, f_v, i_v, s_f, s_i):
        h_f, h_i = handles(k, f_v, i_v, s_f, s_i)
        h_f.start()
        h_i.start()

    def wait(k, f_v, i_v, s_f, s_i):
        h_f, h_i = handles(k, f_v, i_v, s_f, s_i)
        h_f.wait()
        h_i.wait()

    def process(k, i_v, f_v):
        row0 = k * CHUNK

        def one_row(i):
            g = row0 + i
            bid = i_v[pl.ds(i, L)][0]
            sloc = jnp.minimum(jnp.maximum(bid - seg_base, 0), SPW - 1)
            f = [f_v[i, pl.ds(j * L, L)] for j in range(DL)]
            t = [f[j] * Ws[j] for j in range(DL)]
            while len(t) > 1:
                t = [t[2 * j] + t[2 * j + 1] for j in range(len(t) // 2)]
            sv = jnp.full((L,), jnp.sum(t[0]), jnp.float32) + bsplat
            wv = 1.0 / (1.0 + jnp.exp(-sv))
            act = jnp.where((g >= r_lo) & (g < r_hi), 1.0, 0.0)
            wv = wv * act
            return sloc, [wv * f[j] for j in range(DL)]

        @plsc.parallel_loop(0, CHUNK, unroll=2)
        def single_row(i):
            loc, vals = one_row(i)
            for j in range(DL):
                plsc.addupdate(acc_v.at[loc, pl.ds(j * L, L)], vals[j])

    k_lo = r_lo // CHUNK
    k_hi = jnp.maximum(lax.div(r_hi + CHUNK - 1, CHUNK), k_lo)
    npairs = (k_hi - k_lo + 1) // 2

    start(k_lo, f0_v, ids0_v, sf0, si0)

    def pair_body(p, carry):
        k0 = k_lo + 2 * p
        start(k0 + 1, f1_v, ids1_v, sf1, si1)
        wait(k0, f0_v, ids0_v, sf0, si0)
        process(k0, ids0_v, f0_v)
        start(k0 + 2, f0_v, ids0_v, sf0, si0)
        wait(k0 + 1, f1_v, ids1_v, sf1, si1)
        process(k0 + 1, ids1_v, f1_v)
        return carry

    lax.fori_loop(0, npairs, pair_body, 0)

    wait(k_lo, f0_v, ids0_v, sf0, si0)

    @pl.when(wid < NW - 1)
    def _():
        pltpu.sync_copy(acc_v, out_hbm.at[pl.ds(seg_base, SPW)])

    @pl.when(wid == NW - 1)
    def _():
        pltpu.sync_copy(acc_v.at[pl.ds(0, SPW_LAST)],
                        out_hbm.at[pl.ds(seg_base, SPW_LAST)])


_sc_call = functools.partial(
    pl.kernel,
    mesh=plsc.VectorSubcoreMesh(core_axis_name="c", subcore_axis_name="s"),
    compiler_params=pltpu.CompilerParams(needs_layout_passes=False),
    out_type=jax.ShapeDtypeStruct((NSEG, D), jnp.float32),
    scratch_types=[
        pltpu.VMEM((D + L,), jnp.float32),
        pltpu.VMEM((ROWB_PAD,), jnp.int32),
        pltpu.VMEM((CHUNK + L,), jnp.int32),
        pltpu.VMEM((CHUNK + L,), jnp.int32),
        pltpu.VMEM((CHUNK, D), jnp.float32),
        pltpu.VMEM((CHUNK, D), jnp.float32),
        pltpu.VMEM((SPW, D), jnp.float32),
        pltpu.SemaphoreType.DMA,
        pltpu.SemaphoreType.DMA,
        pltpu.SemaphoreType.DMA,
        pltpu.SemaphoreType.DMA,
        pltpu.SemaphoreType.REGULAR,
    ],
)(_sc_body)


def kernel(feats, batch, W, b):
    seg_bounds = jnp.arange(NW + 1, dtype=jnp.int32) * SPW
    rowb = jnp.sum(batch[None, :] < seg_bounds[:, None], axis=1,
                   dtype=jnp.int32)
    rowb = jnp.concatenate(
        [rowb, jnp.full((ROWB_PAD - NW - 1,), N_ROWS, jnp.int32)])
    wb = jnp.concatenate([W[:, 0], jnp.full((L,), b[0], jnp.float32)])
    return _sc_call(feats, batch, rowb, wb)


# speedup vs baseline: 1.0845x; 1.0845x over previous
import functools

import jax
import jax.numpy as jnp
from jax import lax
from jax.experimental import pallas as pl
from jax.experimental.pallas import tpu as pltpu
from jax.experimental.pallas import tpu_sc as plsc

N_ROWS = 160000
D = 256
NSEG = 10000
L = 16
DL = D // L
NW = 32
SPW = 320
SPW_LAST = NSEG - (NW - 1) * SPW
CHUNK = 80
MAXK = N_ROWS // CHUNK - 1
ROWB_PAD = 48


def _sc_body(feats_hbm, ids_hbm, rowb_hbm, wb_hbm, out_hbm,
             wb_v, rowb_v, ids0_v, ids1_v, f0_v, f1_v, acc_v,
             sf0, sf1, si0, si1, fsem):
    cid = lax.axis_index("c")
    sid = lax.axis_index("s")
    wid = sid * 2 + cid

    pltpu.sync_copy(rowb_hbm, rowb_v)
    pltpu.sync_copy(wb_hbm, wb_v)

    r_lo = rowb_v[pl.ds(wid, L)][0]
    r_hi = rowb_v[pl.ds(wid + 1, L)][0]
    seg_base = wid * SPW

    Ws = [wb_v[pl.ds(j * L, L)] for j in range(DL)]
    bsplat = wb_v[pl.ds(D, L)]

    zero = jnp.zeros((L,), jnp.float32)

    def zrow(r, carry):
        for j in range(DL):
            acc_v[r, pl.ds(j * L, L)] = zero
        return carry

    lax.fori_loop(0, SPW, zrow, 0)

    def handles(k, f_v, i_v, s_f, s_i):
        kc = jnp.minimum(k, MAXK)
        row0 = kc * CHUNK
        h_f = pltpu.make_async_copy(
            feats_hbm.at[pl.ds(row0, CHUNK)], f_v, s_f)
        h_i = pltpu.make_async_copy(
            ids_hbm.at[pl.ds(row0, CHUNK)], i_v.at[pl.ds(0, CHUNK)], s_i)
        return h_f, h_i

    def start(k, f_v, i_v, s_f, s_i):
        h_f, h_i = handles(k, f_v, i_v, s_f, s_i)
        h_f.start()
        h_i.start()

    def wait(k, f_v, i_v, s_f, s_i):
        h_f, h_i = handles(k, f_v, i_v, s_f, s_i)
        h_f.wait()
        h_i.wait()

    def process(k, i_v, f_v):
        row0 = k * CHUNK

        def one_row(i):
            g = row0 + i
            bid = i_v[pl.ds(i, L)][0]
            sloc = jnp.minimum(jnp.maximum(bid - seg_base, 0), SPW - 1)
            f = [f_v[i, pl.ds(j * L, L)] for j in range(DL)]
            t = [f[j] * Ws[j] for j in range(DL)]
            while len(t) > 1:
                t = [t[2 * j] + t[2 * j + 1] for j in range(len(t) // 2)]
            sv = jnp.full((L,), jnp.sum(t[0]), jnp.float32) + bsplat
            wv = 1.0 / (1.0 + jnp.exp(-sv))
            act = jnp.where((g >= r_lo) & (g < r_hi), 1.0, 0.0)
            wv = wv * act
            f2 = [f_v[i, pl.ds(j * L, L)] for j in range(DL)]
            return sloc, [wv * f2[j] for j in range(DL)]

        @plsc.parallel_loop(0, CHUNK)
        def single_row(i):
            loc, vals = one_row(i)
            for j in range(DL):
                plsc.addupdate(acc_v.at[loc, pl.ds(j * L, L)], vals[j])

    k_lo = r_lo // CHUNK
    k_hi = jnp.maximum(lax.div(r_hi + CHUNK - 1, CHUNK), k_lo)
    npairs = (k_hi - k_lo + 1) // 2

    start(k_lo, f0_v, ids0_v, sf0, si0)

    def pair_body(p, carry):
        k0 = k_lo + 2 * p
        start(k0 + 1, f1_v, ids1_v, sf1, si1)
        wait(k0, f0_v, ids0_v, sf0, si0)
        process(k0, ids0_v, f0_v)
        start(k0 + 2, f0_v, ids0_v, sf0, si0)
        wait(k0 + 1, f1_v, ids1_v, sf1, si1)
        process(k0 + 1, ids1_v, f1_v)
        return carry

    lax.fori_loop(0, npairs, pair_body, 0)

    wait(k_lo, f0_v, ids0_v, sf0, si0)

    @pl.when(wid < NW - 1)
    def _():
        pltpu.sync_copy(acc_v, out_hbm.at[pl.ds(seg_base, SPW)])

    @pl.when(wid == NW - 1)
    def _():
        pltpu.sync_copy(acc_v.at[pl.ds(0, SPW_LAST)],
                        out_hbm.at[pl.ds(seg_base, SPW_LAST)])


_sc_call = functools.partial(
    pl.kernel,
    mesh=plsc.VectorSubcoreMesh(core_axis_name="c", subcore_axis_name="s"),
    compiler_params=pltpu.CompilerParams(needs_layout_passes=False),
    out_type=jax.ShapeDtypeStruct((NSEG, D), jnp.float32),
    scratch_types=[
        pltpu.VMEM((D + L,), jnp.float32),
        pltpu.VMEM((ROWB_PAD,), jnp.int32),
        pltpu.VMEM((CHUNK + L,), jnp.int32),
        pltpu.VMEM((CHUNK + L,), jnp.int32),
        pltpu.VMEM((CHUNK, D), jnp.float32),
        pltpu.VMEM((CHUNK, D), jnp.float32),
        pltpu.VMEM((SPW, D), jnp.float32),
        pltpu.SemaphoreType.DMA,
        pltpu.SemaphoreType.DMA,
        pltpu.SemaphoreType.DMA,
        pltpu.SemaphoreType.DMA,
        pltpu.SemaphoreType.REGULAR,
    ],
)(_sc_body)


def kernel(feats, batch, W, b):
    seg_bounds = jnp.arange(NW + 1, dtype=jnp.int32) * SPW
    rowb = jnp.sum(batch[None, :] < seg_bounds[:, None], axis=1,
                   dtype=jnp.int32)
    rowb = jnp.concatenate(
        [rowb, jnp.full((ROWB_PAD - NW - 1,), N_ROWS, jnp.int32)])
    wb = jnp.concatenate([W[:, 0], jnp.full((L,), b[0], jnp.float32)])
    return _sc_call(feats, batch, rowb, wb)
